# baseline (speedup 1.0000x reference)
import jax
import jax.numpy as jnp
from jax import lax
from jax.experimental import pallas as pl
from jax.experimental.pallas import tpu as pltpu

B1, B2, V = 32, 8, 100000
ROWS = B1 * B2
R = 32
NROWB = ROWS // R
W = 32768
NCH = (V + W - 1) // W
G = W // 128
NEG_INF = float("-inf")


def _gumbel_onehot_body(logits_ref, u_ref, out_ref, bv_ref, bi_ref, fin_ref):
    r = pl.program_id(0)
    j = pl.program_id(1)

    @pl.when(r > 0)
    def _():
        t = fin_ref[:, 0:1] - j * W
        tg = lax.shift_right_arithmetic(t, 7)
        tl = lax.bitwise_and(t, 127)
        lane = lax.broadcasted_iota(jnp.int32, (R, 128), 1)
        onehot128 = jnp.where(lane == tl, jnp.float32(1.0), jnp.float32(0.0))
        zeros128 = jnp.zeros((R, 128), jnp.float32)
        for g in range(G):
            out_ref[:, g * 128:(g + 1) * 128] = jnp.where(
                tg == g, onehot128, zeros128)

    @pl.when(r < NROWB)
    def _():
        lane = lax.broadcasted_iota(jnp.int32, (R, 128), 1)
        m0 = jnp.where(j == 0, jnp.full((R, 128), NEG_INF, jnp.float32),
                       bv_ref[...])
        i0 = jnp.where(j == 0, jnp.zeros((R, 128), jnp.int32), bi_ref[...])
        base = j * W

        def body(g, carry):
            m, i = carry
            off = g * 128
            u = u_ref[:, pl.dslice(off, 128)]
            lg = logits_ref[:, pl.dslice(off, 128)]
            y = lg - jnp.log(1e-20 - jnp.log(u + 1e-20))
            cols = lane + (base + off)
            y = jnp.where(cols < V, y, NEG_INF)
            upd = y > m
            return jnp.where(upd, y, m), jnp.where(upd, cols, i)

        m, i = lax.fori_loop(0, G, body, (m0, i0), unroll=4)
        bv_ref[...] = m
        bi_ref[...] = i

        @pl.when(j == NCH - 1)
        def _():
            mrow = jnp.max(m, axis=1, keepdims=True)
            cand = jnp.where(m == mrow, i, jnp.int32(2**30))
            fin_ref[:, 0:1] = jnp.min(cand, axis=1, keepdims=True)


def kernel(logits, u):
    logits2d = logits.reshape(ROWS, V)
    u2d = u.reshape(ROWS, V)
    out = pl.pallas_call(
        _gumbel_onehot_body,
        grid=(NROWB + 1, NCH),
        in_specs=[
            pl.BlockSpec((R, W), lambda r, j: (jnp.minimum(r, NROWB - 1),
                                               jnp.where(r < NROWB, j, NCH - 1))),
            pl.BlockSpec((R, W), lambda r, j: (jnp.minimum(r, NROWB - 1),
                                               jnp.where(r < NROWB, j, NCH - 1))),
        ],
        out_specs=pl.BlockSpec((R, W), lambda r, j: (jnp.maximum(r - 1, 0),
                                                     jnp.where(r > 0, j, 0))),
        out_shape=jax.ShapeDtypeStruct((ROWS, V), jnp.float32),
        scratch_shapes=[
            pltpu.VMEM((R, 128), jnp.float32),
            pltpu.VMEM((R, 128), jnp.int32),
            pltpu.VMEM((R, 128), jnp.int32),
        ],
        compiler_params=pltpu.CompilerParams(
            dimension_semantics=("arbitrary", "arbitrary"),
        ),
    )(logits2d, u2d)
    return out.reshape(B1, B2, V)


# speedup vs baseline: 1.0431x; 1.0431x over previous
import jax
import jax.numpy as jnp
from jax import lax
from jax.experimental import pallas as pl
from jax.experimental.pallas import tpu as pltpu

B1, B2, V = 32, 8, 100000
ROWS = B1 * B2
R = 32
NROWB = ROWS // R
W = 12544
NCH = (V + W - 1) // W
G = W // 128
NEG_INF = float("-inf")


def _gumbel_onehot_body(logits_ref, u_ref, out_ref, bv_ref, bi_ref, fin_ref):
    r = pl.program_id(0)
    j = pl.program_id(1)

    @pl.when(r > 0)
    def _():
        t = fin_ref[:, 0:1] - j * W
        tg = lax.shift_right_arithmetic(t, 7)
        tl = lax.bitwise_and(t, 127)
        lane = lax.broadcasted_iota(jnp.int32, (R, 128), 1)
        onehot128 = jnp.where(lane == tl, jnp.float32(1.0), jnp.float32(0.0))
        zeros128 = jnp.zeros((R, 128), jnp.float32)
        for g in range(G):
            out_ref[:, g * 128:(g + 1) * 128] = jnp.where(
                tg == g, onehot128, zeros128)

    @pl.when(r < NROWB)
    def _():
        lane = lax.broadcasted_iota(jnp.int32, (R, 128), 1)
        m0 = jnp.where(j == 0, jnp.full((R, 128), NEG_INF, jnp.float32),
                       bv_ref[...])
        i0 = jnp.where(j == 0, jnp.zeros((R, 128), jnp.int32), bi_ref[...])
        base = j * W

        def body(g, carry):
            m, i = carry
            off = g * 128
            u = u_ref[:, pl.dslice(off, 128)]
            lg = logits_ref[:, pl.dslice(off, 128)]
            y = lg - jnp.log(1e-20 - jnp.log(u + 1e-20))
            cols = lane + (base + off)
            y = jnp.where(cols < V, y, NEG_INF)
            upd = y > m
            return jnp.where(upd, y, m), jnp.where(upd, cols, i)

        m, i = lax.fori_loop(0, G, body, (m0, i0), unroll=4)
        bv_ref[...] = m
        bi_ref[...] = i

        @pl.when(j == NCH - 1)
        def _():
            mrow = jnp.max(m, axis=1, keepdims=True)
            cand = jnp.where(m == mrow, i, jnp.int32(2**30))
            fin_ref[:, 0:1] = jnp.min(cand, axis=1, keepdims=True)


def kernel(logits, u):
    logits2d = logits.reshape(ROWS, V)
    u2d = u.reshape(ROWS, V)
    out = pl.pallas_call(
        _gumbel_onehot_body,
        grid=(NROWB + 1, NCH),
        in_specs=[
            pl.BlockSpec((R, W), lambda r, j: (jnp.minimum(r, NROWB - 1),
                                               jnp.where(r < NROWB, j, NCH - 1))),
            pl.BlockSpec((R, W), lambda r, j: (jnp.minimum(r, NROWB - 1),
                                               jnp.where(r < NROWB, j, NCH - 1))),
        ],
        out_specs=pl.BlockSpec((R, W), lambda r, j: (jnp.maximum(r - 1, 0),
                                                     jnp.where(r > 0, j, 0))),
        out_shape=jax.ShapeDtypeStruct((ROWS, V), jnp.float32),
        scratch_shapes=[
            pltpu.VMEM((R, 128), jnp.float32),
            pltpu.VMEM((R, 128), jnp.int32),
            pltpu.VMEM((R, 128), jnp.int32),
        ],
        compiler_params=pltpu.CompilerParams(
            dimension_semantics=("arbitrary", "arbitrary"),
        ),
    )(logits2d, u2d)
    return out.reshape(B1, B2, V)


# speedup vs baseline: 1.2602x; 1.2082x over previous
import jax
import jax.numpy as jnp
from jax import lax
from jax.experimental import pallas as pl
from jax.experimental.pallas import tpu as pltpu

B1, B2, V = 32, 8, 100000
ROWS = B1 * B2
R = 64
NROWB = ROWS // R
W = 12544
NCH = (V + W - 1) // W
G = W // 128
NEG_INF = float("-inf")


def _gumbel_onehot_body(logits_ref, u_ref, out_ref, bv_ref, bi_ref, fin_ref):
    r = pl.program_id(0)
    j = pl.program_id(1)

    @pl.when(r > 0)
    def _():
        t = fin_ref[:, 0:1] - j * W
        tg = lax.shift_right_arithmetic(t, 7)
        tl = lax.bitwise_and(t, 127)
        lane = lax.broadcasted_iota(jnp.int32, (R, 128), 1)
        onehot128 = jnp.where(lane == tl, jnp.float32(1.0), jnp.float32(0.0))
        zeros128 = jnp.zeros((R, 128), jnp.float32)
        for g in range(G):
            out_ref[:, g * 128:(g + 1) * 128] = jnp.where(
                tg == g, onehot128, zeros128)

    @pl.when(r < NROWB)
    def _():
        lane = lax.broadcasted_iota(jnp.int32, (R, 128), 1)
        m0 = jnp.where(j == 0, jnp.full((R, 128), NEG_INF, jnp.float32),
                       bv_ref[...])
        i0 = jnp.where(j == 0, jnp.zeros((R, 128), jnp.int32), bi_ref[...])
        base = j * W

        def body(g, carry):
            m, i = carry
            off = g * 128
            u = u_ref[:, pl.dslice(off, 128)]
            lg = logits_ref[:, pl.dslice(off, 128)]
            y = lg - jnp.log(1e-20 - jnp.log(u + 1e-20))
            cols = lane + (base + off)
            y = jnp.where(cols < V, y, NEG_INF)
            upd = y > m
            return jnp.where(upd, y, m), jnp.where(upd, cols, i)

        m, i = lax.fori_loop(0, G, body, (m0, i0), unroll=4)
        bv_ref[...] = m
        bi_ref[...] = i

        @pl.when(j == NCH - 1)
        def _():
            mrow = jnp.max(m, axis=1, keepdims=True)
            cand = jnp.where(m == mrow, i, jnp.int32(2**30))
            fin_ref[:, 0:1] = jnp.min(cand, axis=1, keepdims=True)


def kernel(logits, u):
    logits2d = logits.reshape(ROWS, V)
    u2d = u.reshape(ROWS, V)
    out = pl.pallas_call(
        _gumbel_onehot_body,
        grid=(NROWB + 1, NCH),
        in_specs=[
            pl.BlockSpec((R, W), lambda r, j: (jnp.minimum(r, NROWB - 1),
                                               jnp.where(r < NROWB, j, NCH - 1))),
            pl.BlockSpec((R, W), lambda r, j: (jnp.minimum(r, NROWB - 1),
                                               jnp.where(r < NROWB, j, NCH - 1))),
        ],
        out_specs=pl.BlockSpec((R, W), lambda r, j: (jnp.maximum(r - 1, 0),
                                                     jnp.where(r > 0, j, 0))),
        out_shape=jax.ShapeDtypeStruct((ROWS, V), jnp.float32),
        scratch_shapes=[
            pltpu.VMEM((R, 128), jnp.float32),
            pltpu.VMEM((R, 128), jnp.int32),
            pltpu.VMEM((R, 128), jnp.int32),
        ],
        compiler_params=pltpu.CompilerParams(
            dimension_semantics=("arbitrary", "arbitrary"),
        ),
    )(logits2d, u2d)
    return out.reshape(B1, B2, V)


# speedup vs baseline: 1.3387x; 1.0623x over previous
import jax
import jax.numpy as jnp
from jax import lax
from jax.experimental import pallas as pl
from jax.experimental.pallas import tpu as pltpu

B1, B2, V = 32, 8, 100000
ROWS = B1 * B2
R = 128
NROWB = ROWS // R
W = 12544
NCH = (V + W - 1) // W
G = W // 128
NEG_INF = float("-inf")


def _gumbel_onehot_body(logits_ref, u_ref, out_ref, bv_ref, bi_ref, fin_ref):
    r = pl.program_id(0)
    j = pl.program_id(1)

    @pl.when(r > 0)
    def _():
        t = fin_ref[:, 0:1] - j * W
        tg = lax.shift_right_arithmetic(t, 7)
        tl = lax.bitwise_and(t, 127)
        lane = lax.broadcasted_iota(jnp.int32, (R, 128), 1)
        onehot128 = jnp.where(lane == tl, jnp.float32(1.0), jnp.float32(0.0))
        zeros128 = jnp.zeros((R, 128), jnp.float32)
        for g in range(G):
            out_ref[:, g * 128:(g + 1) * 128] = jnp.where(
                tg == g, onehot128, zeros128)

    @pl.when(r < NROWB)
    def _():
        lane = lax.broadcasted_iota(jnp.int32, (R, 128), 1)
        m0 = jnp.where(j == 0, jnp.full((R, 128), NEG_INF, jnp.float32),
                       bv_ref[...])
        i0 = jnp.where(j == 0, jnp.zeros((R, 128), jnp.int32), bi_ref[...])
        base = j * W

        def body(g, carry):
            m, i = carry
            off = g * 128
            u = u_ref[:, pl.dslice(off, 128)]
            lg = logits_ref[:, pl.dslice(off, 128)]
            y = lg - jnp.log(1e-20 - jnp.log(u + 1e-20))
            cols = lane + (base + off)
            y = jnp.where(cols < V, y, NEG_INF)
            upd = y > m
            return jnp.where(upd, y, m), jnp.where(upd, cols, i)

        m, i = lax.fori_loop(0, G, body, (m0, i0), unroll=4)
        bv_ref[...] = m
        bi_ref[...] = i

        @pl.when(j == NCH - 1)
        def _():
            mrow = jnp.max(m, axis=1, keepdims=True)
            cand = jnp.where(m == mrow, i, jnp.int32(2**30))
            fin_ref[:, 0:1] = jnp.min(cand, axis=1, keepdims=True)


def kernel(logits, u):
    logits2d = logits.reshape(ROWS, V)
    u2d = u.reshape(ROWS, V)
    out = pl.pallas_call(
        _gumbel_onehot_body,
        grid=(NROWB + 1, NCH),
        in_specs=[
            pl.BlockSpec((R, W), lambda r, j: (jnp.minimum(r, NROWB - 1),
                                               jnp.where(r < NROWB, j, NCH - 1))),
            pl.BlockSpec((R, W), lambda r, j: (jnp.minimum(r, NROWB - 1),
                                               jnp.where(r < NROWB, j, NCH - 1))),
        ],
        out_specs=pl.BlockSpec((R, W), lambda r, j: (jnp.maximum(r - 1, 0),
                                                     jnp.where(r > 0, j, 0))),
        out_shape=jax.ShapeDtypeStruct((ROWS, V), jnp.float32),
        scratch_shapes=[
            pltpu.VMEM((R, 128), jnp.float32),
            pltpu.VMEM((R, 128), jnp.int32),
            pltpu.VMEM((R, 128), jnp.int32),
        ],
        compiler_params=pltpu.CompilerParams(
            dimension_semantics=("arbitrary", "arbitrary"),
        ),
    )(logits2d, u2d)
    return out.reshape(B1, B2, V)
